# baseline (speedup 1.0000x reference)
import jax
import jax.numpy as jnp
from jax import lax
from jax.experimental import pallas as pl
from jax.experimental.pallas import tpu as pltpu
from jax.experimental.pallas import tpu_sc as plsc

N = 5000
NPAD = 5120
BB_NUM = 100
THR = 0.5
CLIP_MAX = 511.0
B = 4
L = 16
CH = NPAD // L
UNROLL = 4


def _sc_body(c0, c1, r0, r1, r2, r3, a0, a1, a2, a3, out,
             c0v, c1v, r0v, r1v, r2v, r3v, a0v, a1v, a2v, a3v,
             scv, x0v, y0v, x1v, y1v, arv,
             ob0, ob1, ob2, ob3, ob4):
    wid = lax.axis_index("s") * 2 + lax.axis_index("c")
    iota = lax.iota(jnp.int32, L)

    @pl.when(wid < B)
    def _():
        pltpu.sync_copy(c0.at[wid], c0v)
        pltpu.sync_copy(c1.at[wid], c1v)
        pltpu.sync_copy(r0.at[wid], r0v)
        pltpu.sync_copy(r1.at[wid], r1v)
        pltpu.sync_copy(r2.at[wid], r2v)
        pltpu.sync_copy(r3.at[wid], r3v)
        pltpu.sync_copy(a0, a0v)
        pltpu.sync_copy(a1, a1v)
        pltpu.sync_copy(a2, a2v)
        pltpu.sync_copy(a3, a3v)

        def decode(i, _):
            for u in range(UNROLL):
                k = i * UNROLL + u
                s = pl.ds(k * L, L)
                e0 = jnp.exp(c0v[s])
                e1 = jnp.exp(c1v[s])
                gidx = k * L + iota
                scv[s] = jnp.where(gidx < N, e0 / (e0 + e1), -1.0)
                xmin = jnp.maximum(a0v[s] - r0v[s], 0.0)
                ymin = jnp.maximum(a1v[s] - r1v[s], 0.0)
                xmax = jnp.minimum(a2v[s] + r2v[s], CLIP_MAX)
                ymax = jnp.minimum(a3v[s] + r3v[s], CLIP_MAX)
                x0v[s] = xmin
                y0v[s] = ymin
                x1v[s] = xmax
                y1v[s] = ymax
                arv[s] = (jnp.maximum(xmax - xmin, 0.0)
                          * jnp.maximum(ymax - ymin, 0.0))
            return 0

        lax.fori_loop(0, CH // UNROLL, decode, 0)

        def step(t, lasts):
            l0, l1, l2, l3, ls = lasts

            def amx(i, carry):
                rm, gi = carry
                for u in range(UNROLL):
                    k = i * UNROLL + u
                    v = scv[pl.ds(k * L, L)]
                    gt = v > rm
                    rm = jnp.where(gt, v, rm)
                    gi = jnp.where(gt, k * L + iota, gi)
                return rm, gi

            rm, gi = lax.fori_loop(
                0, CH // UNROLL, amx,
                (jnp.full((L,), -2.0, jnp.float32),
                 jnp.zeros((L,), jnp.int32)))
            m, sel = jnp.float32(-2.0), jnp.int32(NPAD)
            for j in range(L):
                v = rm[j]
                g = gi[j]
                better = (v > m) | ((v == m) & (g < sel))
                m = jnp.where(better, v, m)
                sel = jnp.where(better, g, sel)
            valid = m >= 0.0
            sel = jnp.minimum(sel, NPAD - 1)
            cb = (sel // L) * L
            lane = sel % L
            csl = pl.ds(cb, L)

            def pick(ref):
                ch = ref[csl]
                b = ch[0]
                for j in range(1, L):
                    b = jnp.where(lane == j, ch[j], b)
                return b

            b0 = pick(x0v)
            b1 = pick(y0v)
            b2 = pick(x1v)
            b3 = pick(y1v)
            pa = (jnp.maximum(b2 - b0, 0.0) * jnp.maximum(b3 - b1, 0.0))

            def swp(i, _):
                for u in range(UNROLL):
                    k = i * UNROLL + u
                    s = pl.ds(k * L, L)
                    iw = jnp.minimum(x1v[s], b2) - jnp.maximum(x0v[s], b0)
                    ih = jnp.minimum(y1v[s], b3) - jnp.maximum(y0v[s], b1)
                    inter = jnp.maximum(iw, 0.0) * jnp.maximum(ih, 0.0)
                    iou = inter / (arv[s] + pa - inter + 1e-9)
                    scv[s] = jnp.where(iou > THR, -1.0, scv[s])
                return 0

            lax.fori_loop(0, CH // UNROLL, swp, 0)
            scv[csl] = jnp.where(iota == lane, -1.0, scv[csl])
            v0 = jnp.where(valid, b0, l0)
            v1 = jnp.where(valid, b1, l1)
            v2 = jnp.where(valid, b2, l2)
            v3 = jnp.where(valid, b3, l3)
            vs = jnp.where(valid, m, ls)
            ob = pl.ds((t // L) * L, L)
            ot = iota == (t % L)
            ob0[ob] = jnp.where(ot, v0, ob0[ob])
            ob1[ob] = jnp.where(ot, v1, ob1[ob])
            ob2[ob] = jnp.where(ot, v2, ob2[ob])
            ob3[ob] = jnp.where(ot, v3, ob3[ob])
            ob4[ob] = jnp.where(ot, vs, ob4[ob])
            return (v0, v1, v2, v3, vs)

        z = jnp.float32(0.0)
        lax.fori_loop(0, BB_NUM, step, (z, z, z, z, z))
        pltpu.sync_copy(ob0, out.at[wid * 5 + 0])
        pltpu.sync_copy(ob1, out.at[wid * 5 + 1])
        pltpu.sync_copy(ob2, out.at[wid * 5 + 2])
        pltpu.sync_copy(ob3, out.at[wid * 5 + 3])
        pltpu.sync_copy(ob4, out.at[wid * 5 + 4])


_vm = lambda n: pltpu.VMEM((n,), jnp.float32)

_sc_fn = pl.kernel(
    _sc_body,
    out_type=jax.ShapeDtypeStruct((B * 5, 128), jnp.float32),
    mesh=plsc.VectorSubcoreMesh(core_axis_name="c", subcore_axis_name="s"),
    scratch_types=[_vm(NPAD)] * 16 + [_vm(128)] * 5,
)


def kernel(cl, re, anc):
    pad = NPAD - N

    def prep(x):
        return jnp.pad(x, ((0, 0), (0, pad)))

    def prepa(x):
        return jnp.pad(x, (0, pad))

    out = _sc_fn(
        prep(cl[..., 0]), prep(cl[..., 1]),
        prep(re[..., 0]), prep(re[..., 1]),
        prep(re[..., 2]), prep(re[..., 3]),
        prepa(anc[0, :, 0]), prepa(anc[0, :, 1]),
        prepa(anc[0, :, 2]), prepa(anc[0, :, 3])).reshape(B, 5, 128)

    bb = jnp.stack([out[:, 0, :BB_NUM], out[:, 1, :BB_NUM],
                    out[:, 2, :BB_NUM], out[:, 3, :BB_NUM]], axis=-1)
    ffo = out[:, 4, :BB_NUM]
    return bb, ffo


# speedup vs baseline: 1.2610x; 1.2610x over previous
import jax
import jax.numpy as jnp
from jax import lax
from jax.experimental import pallas as pl
from jax.experimental.pallas import tpu as pltpu
from jax.experimental.pallas import tpu_sc as plsc

N = 5000
NPAD = 5120
BB_NUM = 100
THR = 0.5
CLIP_MAX = 511.0
B = 4
L = 16
NSL = 8
SL = NPAD // NSL
SCH = SL // L
UNROLL = 8


def _sc_body(c0, c1, r0, r1, r2, r3, a0, a1, a2, a3, out,
             shared,
             c0v, c1v, x0v, y0v, x1v, y1v, a0v, a1v, a2v, a3v, arv,
             scv, pubv, grpv,
             ob0, ob1, ob2, ob3, ob4):
    cid = lax.axis_index("c")
    sid = lax.axis_index("s")
    img = cid * 2 + sid // NSL
    sl = sid % NSL
    gbase = sl * SL
    iota = lax.iota(jnp.int32, L)

    pltpu.sync_copy(c0.at[img, pl.ds(gbase, SL)], c0v)
    pltpu.sync_copy(c1.at[img, pl.ds(gbase, SL)], c1v)
    pltpu.sync_copy(r0.at[img], x0v)
    pltpu.sync_copy(r1.at[img], y0v)
    pltpu.sync_copy(r2.at[img], x1v)
    pltpu.sync_copy(r3.at[img], y1v)
    pltpu.sync_copy(a0, a0v)
    pltpu.sync_copy(a1, a1v)
    pltpu.sync_copy(a2, a2v)
    pltpu.sync_copy(a3, a3v)

    def decode(i, _):
        for u in range(UNROLL):
            k = i * UNROLL + u
            s = pl.ds(k * L, L)
            xmin = jnp.maximum(a0v[s] - x0v[s], 0.0)
            ymin = jnp.maximum(a1v[s] - y0v[s], 0.0)
            xmax = jnp.minimum(a2v[s] + x1v[s], CLIP_MAX)
            ymax = jnp.minimum(a3v[s] + y1v[s], CLIP_MAX)
            x0v[s] = xmin
            y0v[s] = ymin
            x1v[s] = xmax
            y1v[s] = ymax
            arv[s] = (jnp.maximum(xmax - xmin, 0.0)
                      * jnp.maximum(ymax - ymin, 0.0))
        return 0

    lax.fori_loop(0, (NPAD // L) // UNROLL, decode, 0)

    def sinit(i, carry):
        rm, gi = carry
        for u in range(UNROLL):
            k = i * UNROLL + u
            s = pl.ds(k * L, L)
            e0 = jnp.exp(c0v[s])
            e1 = jnp.exp(c1v[s])
            gidx = gbase + k * L + iota
            v = jnp.where(gidx < N, e0 / (e0 + e1), -1.0)
            scv[s] = v
            gt = v > rm
            rm = jnp.where(gt, v, rm)
            gi = jnp.where(gt, gidx, gi)
        return rm, gi

    rm0, gi0 = lax.fori_loop(0, SCH // UNROLL, sinit,
                             (jnp.full((L,), -2.0, jnp.float32),
                              jnp.zeros((L,), jnp.int32)))

    def step(t, carry):
        rm, gi, l0, l1, l2, l3, ls = carry
        m_loc, s_loc = jnp.float32(-2.0), jnp.int32(NPAD)
        for j in range(L):
            v = rm[j]
            g = gi[j]
            better = (v > m_loc) | ((v == m_loc) & (g < s_loc))
            m_loc = jnp.where(better, v, m_loc)
            s_loc = jnp.where(better, g, s_loc)
        pub = jnp.where(iota == 0, m_loc,
                        jnp.where(iota == 1, s_loc.astype(jnp.float32), 0.0))
        pubv[pl.ds(0, L)] = pub
        pltpu.sync_copy(pubv, shared.at[pl.ds((cid * 16 + sid) * L, L)])
        plsc.subcore_barrier()
        g0 = (cid * 16 + (sid // NSL) * NSL) * L
        pltpu.sync_copy(shared.at[pl.ds(g0, NSL * L)], grpv)
        plsc.subcore_barrier()
        m, sel = jnp.float32(-2.0), jnp.int32(NPAD)
        for j in range(NSL):
            row = grpv[pl.ds(j * L, L)]
            v = row[0]
            g = row[1].astype(jnp.int32)
            better = (v > m) | ((v == m) & (g < sel))
            m = jnp.where(better, v, m)
            sel = jnp.where(better, g, sel)
        valid = m >= 0.0
        sel = jnp.minimum(sel, NPAD - 1)
        cb = (sel // L) * L
        lane = sel % L
        csl = pl.ds(cb, L)

        def pick(ref):
            ch = ref[csl]
            b = ch[0]
            for j in range(1, L):
                b = jnp.where(lane == j, ch[j], b)
            return b

        b0 = pick(x0v)
        b1 = pick(y0v)
        b2 = pick(x1v)
        b3 = pick(y1v)
        pa = jnp.maximum(b2 - b0, 0.0) * jnp.maximum(b3 - b1, 0.0)

        def swp(i, carry):
            nrm, ngi = carry
            for u in range(UNROLL):
                k = i * UNROLL + u
                s = pl.ds(k * L, L)
                gs = pl.ds(gbase + k * L, L)
                iw = jnp.minimum(x1v[gs], b2) - jnp.maximum(x0v[gs], b0)
                ih = jnp.minimum(y1v[gs], b3) - jnp.maximum(y0v[gs], b1)
                inter = jnp.maximum(iw, 0.0) * jnp.maximum(ih, 0.0)
                iou = inter / (arv[gs] + pa - inter + 1e-9)
                gidx = gbase + k * L + iota
                v = jnp.where((iou > THR) | (gidx == sel), -1.0, scv[s])
                scv[s] = v
                gt = v > nrm
                nrm = jnp.where(gt, v, nrm)
                ngi = jnp.where(gt, gidx, ngi)
            return nrm, ngi

        rm, gi = lax.fori_loop(0, SCH // UNROLL, swp,
                               (jnp.full((L,), -2.0, jnp.float32),
                                jnp.zeros((L,), jnp.int32)))
        v0 = jnp.where(valid, b0, l0)
        v1 = jnp.where(valid, b1, l1)
        v2 = jnp.where(valid, b2, l2)
        v3 = jnp.where(valid, b3, l3)
        vs = jnp.where(valid, m, ls)
        ob = pl.ds((t // L) * L, L)
        ot = iota == (t % L)
        ob0[ob] = jnp.where(ot, v0, ob0[ob])
        ob1[ob] = jnp.where(ot, v1, ob1[ob])
        ob2[ob] = jnp.where(ot, v2, ob2[ob])
        ob3[ob] = jnp.where(ot, v3, ob3[ob])
        ob4[ob] = jnp.where(ot, vs, ob4[ob])
        return (rm, gi, v0, v1, v2, v3, vs)

    z = jnp.float32(0.0)
    lax.fori_loop(0, BB_NUM, step, (rm0, gi0, z, z, z, z, z))

    @pl.when(sl == 0)
    def _():
        pltpu.sync_copy(ob0, out.at[img * 5 + 0])
        pltpu.sync_copy(ob1, out.at[img * 5 + 1])
        pltpu.sync_copy(ob2, out.at[img * 5 + 2])
        pltpu.sync_copy(ob3, out.at[img * 5 + 3])
        pltpu.sync_copy(ob4, out.at[img * 5 + 4])


_vm = lambda n: pltpu.VMEM((n,), jnp.float32)

_sc_fn = pl.kernel(
    _sc_body,
    out_type=jax.ShapeDtypeStruct((B * 5, 128), jnp.float32),
    mesh=plsc.VectorSubcoreMesh(core_axis_name="c", subcore_axis_name="s"),
    scratch_types=([pltpu.VMEM_SHARED((2 * 16 * L,), jnp.float32)]
                   + [_vm(SL)] * 2 + [_vm(NPAD)] * 9
                   + [_vm(SL), _vm(L), _vm(NSL * L)] + [_vm(128)] * 5),
)


def kernel(cl, re, anc):
    pad = NPAD - N

    def prep(x):
        return jnp.pad(x, ((0, 0), (0, pad)))

    def prepa(x):
        return jnp.pad(x, (0, pad))

    out = _sc_fn(
        prep(cl[..., 0]), prep(cl[..., 1]),
        prep(re[..., 0]), prep(re[..., 1]),
        prep(re[..., 2]), prep(re[..., 3]),
        prepa(anc[0, :, 0]), prepa(anc[0, :, 1]),
        prepa(anc[0, :, 2]), prepa(anc[0, :, 3])).reshape(B, 5, 128)

    bb = jnp.stack([out[:, 0, :BB_NUM], out[:, 1, :BB_NUM],
                    out[:, 2, :BB_NUM], out[:, 3, :BB_NUM]], axis=-1)
    ffo = out[:, 4, :BB_NUM]
    return bb, ffo


# speedup vs baseline: 1.2879x; 1.0213x over previous
import jax
import jax.numpy as jnp
from jax import lax
from jax.experimental import pallas as pl
from jax.experimental.pallas import tpu as pltpu
from jax.experimental.pallas import tpu_sc as plsc

N = 5000
NPAD = 5120
BB_NUM = 100
THR = 0.5
CLIP_MAX = 511.0
B = 4
L = 16
NSL = 8
SL = NPAD // NSL
SCH = SL // L
UNROLL = 8


def _sc_body(c0, c1, r0, r1, r2, r3, a0, a1, a2, a3, out,
             shared,
             c0v, c1v, x0v, y0v, x1v, y1v, a0v, a1v, a2v, a3v, arv,
             scv, pubv, grpv,
             ob0, ob1, ob2, ob3, ob4):
    cid = lax.axis_index("c")
    sid = lax.axis_index("s")
    img = cid * 2 + sid // NSL
    sl = sid % NSL
    gbase = sl * SL
    iota = lax.iota(jnp.int32, L)

    pltpu.sync_copy(c0.at[img, pl.ds(gbase, SL)], c0v)
    pltpu.sync_copy(c1.at[img, pl.ds(gbase, SL)], c1v)
    pltpu.sync_copy(r0.at[img], x0v)
    pltpu.sync_copy(r1.at[img], y0v)
    pltpu.sync_copy(r2.at[img], x1v)
    pltpu.sync_copy(r3.at[img], y1v)
    pltpu.sync_copy(a0, a0v)
    pltpu.sync_copy(a1, a1v)
    pltpu.sync_copy(a2, a2v)
    pltpu.sync_copy(a3, a3v)

    def decode(i, _):
        for u in range(UNROLL):
            k = i * UNROLL + u
            s = pl.ds(k * L, L)
            xmin = jnp.maximum(a0v[s] - x0v[s], 0.0)
            ymin = jnp.maximum(a1v[s] - y0v[s], 0.0)
            xmax = jnp.minimum(a2v[s] + x1v[s], CLIP_MAX)
            ymax = jnp.minimum(a3v[s] + y1v[s], CLIP_MAX)
            x0v[s] = xmin
            y0v[s] = ymin
            x1v[s] = xmax
            y1v[s] = ymax
            arv[s] = (jnp.maximum(xmax - xmin, 0.0)
                      * jnp.maximum(ymax - ymin, 0.0))
        return 0

    lax.fori_loop(0, (NPAD // L) // UNROLL, decode, 0)

    def sinit(i, carry):
        rm, gi = carry
        for u in range(UNROLL):
            k = i * UNROLL + u
            s = pl.ds(k * L, L)
            e0 = jnp.exp(c0v[s])
            e1 = jnp.exp(c1v[s])
            gidx = gbase + k * L + iota
            v = jnp.where(gidx < N, e0 / (e0 + e1), -1.0)
            scv[s] = v
            gt = v > rm
            rm = jnp.where(gt, v, rm)
            gi = jnp.where(gt, gidx, gi)
        return rm, gi

    rm0, gi0 = lax.fori_loop(0, SCH // UNROLL, sinit,
                             (jnp.full((L,), -2.0, jnp.float32),
                              jnp.zeros((L,), jnp.int32)))

    def step(t, carry):
        rm, gi, l0, l1, l2, l3, ls = carry
        m_loc, s_loc = jnp.float32(-2.0), jnp.int32(NPAD)
        for j in range(L):
            v = rm[j]
            g = gi[j]
            better = (v > m_loc) | ((v == m_loc) & (g < s_loc))
            m_loc = jnp.where(better, v, m_loc)
            s_loc = jnp.where(better, g, s_loc)
        pub = jnp.where(iota == 0, m_loc,
                        jnp.where(iota == 1, s_loc.astype(jnp.float32), 0.0))
        pubv[pl.ds(0, L)] = pub
        boff = (t % 2) * (2 * 16 * L)
        pltpu.sync_copy(pubv,
                        shared.at[pl.ds(boff + (cid * 16 + sid) * L, L)])
        plsc.subcore_barrier()
        g0 = boff + (cid * 16 + (sid // NSL) * NSL) * L
        pltpu.sync_copy(shared.at[pl.ds(g0, NSL * L)], grpv)
        m, sel = jnp.float32(-2.0), jnp.int32(NPAD)
        for j in range(NSL):
            row = grpv[pl.ds(j * L, L)]
            v = row[0]
            g = row[1].astype(jnp.int32)
            better = (v > m) | ((v == m) & (g < sel))
            m = jnp.where(better, v, m)
            sel = jnp.where(better, g, sel)
        valid = m >= 0.0
        sel = jnp.minimum(sel, NPAD - 1)
        cb = (sel // L) * L
        lane = sel % L
        csl = pl.ds(cb, L)

        def pick(ref):
            ch = ref[csl]
            b = ch[0]
            for j in range(1, L):
                b = jnp.where(lane == j, ch[j], b)
            return b

        b0 = pick(x0v)
        b1 = pick(y0v)
        b2 = pick(x1v)
        b3 = pick(y1v)
        pa = jnp.maximum(b2 - b0, 0.0) * jnp.maximum(b3 - b1, 0.0)

        def swp(i, carry):
            nrm, ngi = carry
            for u in range(UNROLL):
                k = i * UNROLL + u
                s = pl.ds(k * L, L)
                gs = pl.ds(gbase + k * L, L)
                iw = jnp.minimum(x1v[gs], b2) - jnp.maximum(x0v[gs], b0)
                ih = jnp.minimum(y1v[gs], b3) - jnp.maximum(y0v[gs], b1)
                inter = jnp.maximum(iw, 0.0) * jnp.maximum(ih, 0.0)
                iou = inter / (arv[gs] + pa - inter + 1e-9)
                gidx = gbase + k * L + iota
                v = jnp.where((iou > THR) | (gidx == sel), -1.0, scv[s])
                scv[s] = v
                gt = v > nrm
                nrm = jnp.where(gt, v, nrm)
                ngi = jnp.where(gt, gidx, ngi)
            return nrm, ngi

        rm, gi = lax.fori_loop(0, SCH // UNROLL, swp,
                               (jnp.full((L,), -2.0, jnp.float32),
                                jnp.zeros((L,), jnp.int32)))
        v0 = jnp.where(valid, b0, l0)
        v1 = jnp.where(valid, b1, l1)
        v2 = jnp.where(valid, b2, l2)
        v3 = jnp.where(valid, b3, l3)
        vs = jnp.where(valid, m, ls)
        ob = pl.ds((t // L) * L, L)
        ot = iota == (t % L)
        ob0[ob] = jnp.where(ot, v0, ob0[ob])
        ob1[ob] = jnp.where(ot, v1, ob1[ob])
        ob2[ob] = jnp.where(ot, v2, ob2[ob])
        ob3[ob] = jnp.where(ot, v3, ob3[ob])
        ob4[ob] = jnp.where(ot, vs, ob4[ob])
        return (rm, gi, v0, v1, v2, v3, vs)

    z = jnp.float32(0.0)
    lax.fori_loop(0, BB_NUM, step, (rm0, gi0, z, z, z, z, z))

    @pl.when(sl == 0)
    def _():
        pltpu.sync_copy(ob0, out.at[img * 5 + 0])
        pltpu.sync_copy(ob1, out.at[img * 5 + 1])
        pltpu.sync_copy(ob2, out.at[img * 5 + 2])
        pltpu.sync_copy(ob3, out.at[img * 5 + 3])
        pltpu.sync_copy(ob4, out.at[img * 5 + 4])


_vm = lambda n: pltpu.VMEM((n,), jnp.float32)

_sc_fn = pl.kernel(
    _sc_body,
    out_type=jax.ShapeDtypeStruct((B * 5, 128), jnp.float32),
    mesh=plsc.VectorSubcoreMesh(core_axis_name="c", subcore_axis_name="s"),
    scratch_types=([pltpu.VMEM_SHARED((2 * 2 * 16 * L,), jnp.float32)]
                   + [_vm(SL)] * 2 + [_vm(NPAD)] * 9
                   + [_vm(SL), _vm(L), _vm(NSL * L)] + [_vm(128)] * 5),
)


def kernel(cl, re, anc):
    pad = NPAD - N

    def prep(x):
        return jnp.pad(x, ((0, 0), (0, pad)))

    def prepa(x):
        return jnp.pad(x, (0, pad))

    out = _sc_fn(
        prep(cl[..., 0]), prep(cl[..., 1]),
        prep(re[..., 0]), prep(re[..., 1]),
        prep(re[..., 2]), prep(re[..., 3]),
        prepa(anc[0, :, 0]), prepa(anc[0, :, 1]),
        prepa(anc[0, :, 2]), prepa(anc[0, :, 3])).reshape(B, 5, 128)

    bb = jnp.stack([out[:, 0, :BB_NUM], out[:, 1, :BB_NUM],
                    out[:, 2, :BB_NUM], out[:, 3, :BB_NUM]], axis=-1)
    ffo = out[:, 4, :BB_NUM]
    return bb, ffo
